# baseline (speedup 1.0000x reference)
import functools

import jax
import jax.numpy as jnp
from jax import lax
from jax.experimental import pallas as pl
from jax.experimental.pallas import tpu as pltpu
from jax.experimental.pallas import tpu_sc as plsc

N = 10000
NP = 10240
E = 320000
F = 128
H = 64
NC = 2
NS = 16
NW = NC * NS
C = 128
NCH = 80
E_W = NCH * C
EP = NW * E_W
K = 5
NG = NCH // K
R_T = NP // NS
R_C = 128
NRC = R_T // R_C

_f32 = jnp.float32
_mesh = plsc.VectorSubcoreMesh(
    core_axis_name="c", subcore_axis_name="s", num_cores=NC, num_subcores=NS
)
_sc_params = pltpu.CompilerParams(use_tc_tiling_on_sc=False)


@functools.partial(
    pl.kernel,
    out_type=jax.ShapeDtypeStruct((NC, NP, 16), _f32),
    mesh=_mesh,
    compiler_params=_sc_params,
    scratch_types=[
        pltpu.VMEM_SHARED((NP, 16), _f32),
        pltpu.VMEM((R_C, 16), _f32),
        pltpu.VMEM((C, 16), _f32),
        pltpu.VMEM((NCH, C), jnp.int32),
        pltpu.SemaphoreType.DMA,
    ],
)
def _sc_degree(dst_hbm, out_hbm, acc_sp, zbuf, ones_v, idx_d, sem):
    c = lax.axis_index("c")
    s = lax.axis_index("s")
    wid = s * NC + c

    def _zero_row(i, _):
        zbuf[i, :] = jnp.zeros((16,), _f32)
        ones_v[i, :] = jnp.ones((16,), _f32)
        return 0

    lax.fori_loop(0, R_C, _zero_row, 0)

    pltpu.sync_copy(dst_hbm.at[wid], idx_d)
    for r in range(NRC):
        pltpu.sync_copy(zbuf, acc_sp.at[pl.ds(s * R_T + r * R_C, R_C), :])
    plsc.subcore_barrier()

    def _group(g, _):
        descs = [
            pltpu.async_copy(ones_v, acc_sp.at[idx_d.at[g * 2 * K + b]], sem,
                             add=True)
            for b in range(2 * K)
        ]
        for d in descs:
            d.wait()
        return 0

    lax.fori_loop(0, NCH // (2 * K), _group, 0)
    plsc.subcore_barrier()

    for r in range(NRC):
        off = s * R_T + r * R_C
        pltpu.sync_copy(acc_sp.at[pl.ds(off, R_C), :], zbuf)
        pltpu.sync_copy(zbuf, out_hbm.at[c, pl.ds(off, R_C), :])


@functools.partial(
    pl.kernel,
    out_type=jax.ShapeDtypeStruct((NC, NP, H), _f32),
    mesh=_mesh,
    compiler_params=_sc_params,
    scratch_types=[
        pltpu.VMEM_SHARED((NP, H), _f32),
        pltpu.VMEM((R_C, H), _f32),
        pltpu.VMEM((K, C, H), _f32),
        pltpu.VMEM((NCH, C), jnp.int32),
        pltpu.VMEM((NCH, C), jnp.int32),
        pltpu.SemaphoreType.DMA,
        pltpu.SemaphoreType.DMA,
    ],
)
def _sc_edge_pass(hs_hbm, src_hbm, dst_hbm, out_hbm,
                  acc_sp, zbuf, rows, idx_s, idx_d, gsem, ssem):
    c = lax.axis_index("c")
    s = lax.axis_index("s")
    wid = s * NC + c

    def _zero_row(i, _):
        for k in range(H // 16):
            zbuf[i, pl.ds(k * 16, 16)] = jnp.zeros((16,), _f32)
        return 0

    lax.fori_loop(0, R_C, _zero_row, 0)

    pltpu.sync_copy(src_hbm.at[wid], idx_s)
    pltpu.sync_copy(dst_hbm.at[wid], idx_d)

    zds = [
        pltpu.async_copy(zbuf, acc_sp.at[pl.ds(s * R_T + r * R_C, R_C), :], ssem)
        for r in range(NRC)
    ]
    for d in zds:
        d.wait()
    plsc.subcore_barrier()

    def _group(g, _):
        gd = [
            pltpu.async_copy(hs_hbm.at[idx_s.at[g * K + b]], rows.at[b], gsem)
            for b in range(K)
        ]
        for d in gd:
            d.wait()
        sd = [
            pltpu.async_copy(rows.at[b], acc_sp.at[idx_d.at[g * K + b]], ssem,
                             add=True)
            for b in range(K)
        ]
        for d in sd:
            d.wait()
        return 0

    lax.fori_loop(0, NG, _group, 0)
    plsc.subcore_barrier()

    dds = [
        pltpu.async_copy(acc_sp.at[pl.ds(s * R_T + r * R_C, R_C), :],
                         rows.at[r], gsem)
        for r in range(NRC)
    ]
    for d in dds:
        d.wait()
    ods = [
        pltpu.async_copy(rows.at[r],
                         out_hbm.at[c, pl.ds(s * R_T + r * R_C, R_C), :], ssem)
        for r in range(NRC)
    ]
    for d in ods:
        d.wait()


_BLK = 1000


def _dinv_block(degp_ref):
    deg = degp_ref[0, :, 0:1] + degp_ref[1, :, 0:1] + 1.0
    return lax.rsqrt(deg)


def _tc1_body(x_ref, w_ref, degp_ref, o_ref):
    h = jnp.dot(x_ref[...], w_ref[...], preferred_element_type=_f32)
    o_ref[...] = h * _dinv_block(degp_ref)


def _tc2_body(tp_ref, hs_ref, degp_ref, b_ref, o_ref):
    dinv = _dinv_block(degp_ref)
    a = (tp_ref[0] + tp_ref[1] + hs_ref[...]) * dinv + b_ref[...]
    o_ref[...] = jnp.maximum(a, 0.0) * dinv


def _tc3_body(tp_ref, hs_ref, degp_ref, w_ref, b_ref, o_ref):
    y = (tp_ref[0] + tp_ref[1] + hs_ref[...]) * _dinv_block(degp_ref)
    o_ref[...] = (
        jnp.dot(y, w_ref[...], preferred_element_type=_f32) + b_ref[...]
    )


def _rows_spec(width):
    return pl.BlockSpec((_BLK, width), lambda i: (i, 0))


def _part_spec(width):
    return pl.BlockSpec((NC, _BLK, width), lambda i: (0, i, 0))


def _full_spec(shape):
    return pl.BlockSpec(shape, lambda i: tuple(0 for _ in shape))


def _tc1(x, W1, degp):
    return pl.pallas_call(
        _tc1_body,
        grid=(N // _BLK,),
        in_specs=[_rows_spec(F), _full_spec((F, H)), _part_spec(16)],
        out_specs=_rows_spec(H),
        out_shape=jax.ShapeDtypeStruct((NP, H), _f32),
    )(x, W1, degp)


def _tc2(t1p, hs1, degp, b1):
    return pl.pallas_call(
        _tc2_body,
        grid=(N // _BLK,),
        in_specs=[_part_spec(H), _rows_spec(H), _part_spec(16), _full_spec((1, H))],
        out_specs=_rows_spec(H),
        out_shape=jax.ShapeDtypeStruct((NP, H), _f32),
    )(t1p, hs1, degp, b1)


def _tc3(t2p, hs2, degp, W2, b2):
    return pl.pallas_call(
        _tc3_body,
        grid=(N // _BLK,),
        in_specs=[
            _part_spec(H),
            _rows_spec(H),
            _part_spec(16),
            _full_spec((H, F)),
            _full_spec((1, F)),
        ],
        out_specs=_rows_spec(F),
        out_shape=jax.ShapeDtypeStruct((N, F), _f32),
    )(t2p, hs2, degp, W2, b2)


def kernel(x, edge_index, W1, b1, W2, b2):
    ei = edge_index.astype(jnp.int32)
    pad = jnp.full((2, EP - E), N, jnp.int32)
    eip = jnp.concatenate([ei, pad], axis=1).reshape(2, NW, NCH, C)
    src, dst = eip[0], eip[1]

    degp = _sc_degree(dst)
    hs1 = _tc1(x, W1, degp)
    t1p = _sc_edge_pass(hs1, src, dst)
    hs2 = _tc2(t1p, hs1, degp, b1.reshape(1, H))
    t2p = _sc_edge_pass(hs2, src, dst)
    return _tc3(t2p, hs2, degp, W2, b2.reshape(1, F))


# speedup vs baseline: 2.3216x; 2.3216x over previous
import functools

import jax
import jax.numpy as jnp
from jax import lax
from jax.experimental import pallas as pl
from jax.experimental.pallas import tpu as pltpu
from jax.experimental.pallas import tpu_sc as plsc

N = 10000
NP = 10240
E = 320000
F = 128
H = 64
NC = 2
NS = 16
NW = NC * NS
C = 128
NCH = 160
E_T = NCH * C
EP = NS * E_T
HH = H // 2
K = 8
NG = NCH // K
R_T = NP // NS
R_C = 128
NRC = R_T // R_C

_f32 = jnp.float32
_mesh = plsc.VectorSubcoreMesh(
    core_axis_name="c", subcore_axis_name="s", num_cores=NC, num_subcores=NS
)
_sc_params = pltpu.CompilerParams(use_tc_tiling_on_sc=False)


NCH_D = NCH // 2


@functools.partial(
    pl.kernel,
    out_type=jax.ShapeDtypeStruct((NC, NP, 16), _f32),
    mesh=_mesh,
    compiler_params=_sc_params,
    scratch_types=[
        pltpu.VMEM_SHARED((NP, 16), _f32),
        pltpu.VMEM((R_C, 16), _f32),
        pltpu.VMEM((C, 16), _f32),
        pltpu.VMEM((NCH_D, C), jnp.int32),
        pltpu.SemaphoreType.DMA,
    ],
)
def _sc_degree(dst_hbm, out_hbm, acc_sp, zbuf, ones_v, idx_d, sem):
    c = lax.axis_index("c")
    s = lax.axis_index("s")

    def _zero_row(i, _):
        zbuf[i, :] = jnp.zeros((16,), _f32)
        ones_v[i, :] = jnp.ones((16,), _f32)
        return 0

    lax.fori_loop(0, R_C, _zero_row, 0)

    pltpu.sync_copy(dst_hbm.at[s, pl.ds(c * NCH_D, NCH_D), :], idx_d)
    for r in range(NRC):
        pltpu.sync_copy(zbuf, acc_sp.at[pl.ds(s * R_T + r * R_C, R_C), :])
    plsc.subcore_barrier()

    def _group(g, _):
        descs = [
            pltpu.async_copy(ones_v, acc_sp.at[idx_d.at[g * 2 * K + b]], sem,
                             add=True)
            for b in range(2 * K)
        ]
        for d in descs:
            d.wait()
        return 0

    lax.fori_loop(0, NCH_D // (2 * K), _group, 0)
    plsc.subcore_barrier()

    for r in range(NRC):
        off = s * R_T + r * R_C
        pltpu.sync_copy(acc_sp.at[pl.ds(off, R_C), :], zbuf)
        pltpu.sync_copy(zbuf, out_hbm.at[c, pl.ds(off, R_C), :])


@functools.partial(
    pl.kernel,
    out_type=jax.ShapeDtypeStruct((NP, H), _f32),
    mesh=_mesh,
    compiler_params=_sc_params,
    scratch_types=[
        pltpu.VMEM_SHARED((NP, HH), _f32),
        pltpu.VMEM_SHARED((NP, HH), _f32),
        pltpu.VMEM((R_C, HH), _f32),
        pltpu.VMEM((K, C, HH), _f32),
        pltpu.VMEM((NCH, C), jnp.int32),
        pltpu.VMEM((NCH, C), jnp.int32),
        pltpu.SemaphoreType.DMA,
        pltpu.SemaphoreType.DMA,
    ],
)
def _sc_edge_pass(hs_hbm, src_hbm, dst_hbm, out_hbm,
                  hs_sp, acc_sp, zbuf, rows, idx_s, idx_d, gsem, ssem):
    c = lax.axis_index("c")
    s = lax.axis_index("s")
    col = c * HH

    def _zero_row(i, _):
        for k in range(HH // 16):
            zbuf[i, pl.ds(k * 16, 16)] = jnp.zeros((16,), _f32)
        return 0

    lax.fori_loop(0, R_C, _zero_row, 0)

    pltpu.sync_copy(src_hbm.at[s], idx_s)
    pltpu.sync_copy(dst_hbm.at[s], idx_d)

    for r in range(NRC):
        off = s * R_T + r * R_C
        pltpu.sync_copy(zbuf, acc_sp.at[pl.ds(off, R_C), :])
        pltpu.sync_copy(hs_hbm.at[pl.ds(off, R_C), pl.ds(col, HH)], rows.at[0])
        pltpu.sync_copy(rows.at[0], hs_sp.at[pl.ds(off, R_C), :])
    plsc.subcore_barrier()

    def _group(g, _):
        gd = [
            pltpu.async_copy(hs_sp.at[idx_s.at[g * K + b]], rows.at[b], gsem)
            for b in range(K)
        ]
        sd = []
        for b in range(K):
            gd[b].wait()
            sd.append(
                pltpu.async_copy(rows.at[b], acc_sp.at[idx_d.at[g * K + b]],
                                 ssem, add=True)
            )
        for d in sd:
            d.wait()
        return 0

    lax.fori_loop(0, NG, _group, 0)
    plsc.subcore_barrier()

    for r in range(NRC):
        off = s * R_T + r * R_C
        pltpu.sync_copy(acc_sp.at[pl.ds(off, R_C), :], zbuf)
        pltpu.sync_copy(zbuf, out_hbm.at[pl.ds(off, R_C), pl.ds(col, HH)])


_BLK = 1000


def _dinv_block(degp_ref):
    deg = degp_ref[0, :, 0:1] + degp_ref[1, :, 0:1] + 1.0
    return lax.rsqrt(deg)


def _tc1_body(x_ref, w_ref, degp_ref, o_ref):
    h = jnp.dot(x_ref[...], w_ref[...], preferred_element_type=_f32)
    o_ref[...] = h * _dinv_block(degp_ref)


def _tc2_body(tp_ref, hs_ref, degp_ref, b_ref, o_ref):
    dinv = _dinv_block(degp_ref)
    a = (tp_ref[...] + hs_ref[...]) * dinv + b_ref[...]
    o_ref[...] = jnp.maximum(a, 0.0) * dinv


def _tc3_body(tp_ref, hs_ref, degp_ref, w_ref, b_ref, o_ref):
    y = (tp_ref[...] + hs_ref[...]) * _dinv_block(degp_ref)
    o_ref[...] = (
        jnp.dot(y, w_ref[...], preferred_element_type=_f32) + b_ref[...]
    )


def _rows_spec(width):
    return pl.BlockSpec((_BLK, width), lambda i: (i, 0))


def _part_spec(width):
    return pl.BlockSpec((NC, _BLK, width), lambda i: (0, i, 0))


def _full_spec(shape):
    return pl.BlockSpec(shape, lambda i: tuple(0 for _ in shape))


def _tc1(x, W1, degp):
    return pl.pallas_call(
        _tc1_body,
        grid=(N // _BLK,),
        in_specs=[_rows_spec(F), _full_spec((F, H)), _part_spec(16)],
        out_specs=_rows_spec(H),
        out_shape=jax.ShapeDtypeStruct((NP, H), _f32),
    )(x, W1, degp)


def _tc2(t1p, hs1, degp, b1):
    return pl.pallas_call(
        _tc2_body,
        grid=(N // _BLK,),
        in_specs=[_rows_spec(H), _rows_spec(H), _part_spec(16), _full_spec((1, H))],
        out_specs=_rows_spec(H),
        out_shape=jax.ShapeDtypeStruct((NP, H), _f32),
    )(t1p, hs1, degp, b1)


def _tc3(t2p, hs2, degp, W2, b2):
    return pl.pallas_call(
        _tc3_body,
        grid=(N // _BLK,),
        in_specs=[
            _rows_spec(H),
            _rows_spec(H),
            _part_spec(16),
            _full_spec((H, F)),
            _full_spec((1, F)),
        ],
        out_specs=_rows_spec(F),
        out_shape=jax.ShapeDtypeStruct((N, F), _f32),
    )(t2p, hs2, degp, W2, b2)


def kernel(x, edge_index, W1, b1, W2, b2):
    ei = edge_index.astype(jnp.int32)
    pad = jnp.full((2, EP - E), N, jnp.int32)
    eip = jnp.concatenate([ei, pad], axis=1).reshape(2, NS, NCH, C)
    src, dst = eip[0], eip[1]

    degp = _sc_degree(dst)
    hs1 = _tc1(x, W1, degp)
    t1p = _sc_edge_pass(hs1, src, dst)
    hs2 = _tc2(t1p, hs1, degp, b1.reshape(1, H))
    t2p = _sc_edge_pass(hs2, src, dst)
    return _tc3(t2p, hs2, degp, W2, b2.reshape(1, F))
